# Initial kernel scaffold; baseline (speedup 1.0000x reference)
import jax
import jax.numpy as jnp
from jax.experimental import pallas as pl


def kernel(ent_embeds, rel_embeds, W_msg, W_self, history_times, query_entities, query_relations, node_ids, edge_src, edge_dst, edge_type, edge_query_rel, edge_query_ent, batch_node_ids):
    raise NotImplementedError("write your pallas kernel here")


# speedup vs baseline: 1.4893x; 1.4893x over previous
import functools

import jax
import jax.numpy as jnp
from jax import lax
from jax.experimental import pallas as pl
from jax.experimental.pallas import tpu as pltpu
from jax.experimental.pallas import tpu_sc as plsc

NC, NS, NW = 2, 16, 32
N_ENT, N_REL, D = 100000, 500, 128
N_NODES, N_EDGES = 10000, 320000
BS, HIST = 100, 50

NHALF = N_NODES // NC
ACC_ROWS = NHALF + 8

EK = 128
E_TROWS = 160
E_CH = 16
E_NCH = E_TROWS // E_CH
E_PAD = NS * E_TROWS * EK

ZROWS = 312
ZTAIL = ACC_ROWS - NS * ZROWS

HCHUNK = 40
H_PAD = 10240
H_TROWS = H_PAD // NW // HCHUNK

BCHUNK = 40
B_PAD = 5120
B_TROWS = B_PAD // NW // BCHUNK

_f32 = jnp.float32


def _mesh():
    return plsc.VectorSubcoreMesh(core_axis_name="c", subcore_axis_name="s")


def _gather_nodes_queries(ent, rel, nid3, qe3, qr3):
    @functools.partial(
        pl.kernel,
        out_type=[
            jax.ShapeDtypeStruct((H_PAD, D), _f32),
            jax.ShapeDtypeStruct((128, D), _f32),
            jax.ShapeDtypeStruct((128, D), _f32),
        ],
        mesh=_mesh(),
        scratch_types=[
            pltpu.VMEM((H_TROWS, HCHUNK), jnp.int32),
            pltpu.VMEM((HCHUNK, D), _f32),
            pltpu.VMEM((1, 8), jnp.int32),
            pltpu.VMEM((8, D), _f32),
        ],
    )
    def k(ent_r, rel_r, nid_r, qe_r, qr_r, h_o, qe_o, qr_o,
          nidx_v, buf_v, qidx_v, qbuf_v):
        c = lax.axis_index("c")
        s = lax.axis_index("s")
        wid = c * NS + s

        pltpu.sync_copy(nid_r.at[wid], nidx_v)

        @pl.loop(0, H_TROWS)
        def _(t):
            pltpu.sync_copy(ent_r.at[nidx_v.at[t]], buf_v)
            pltpu.sync_copy(
                buf_v, h_o.at[pl.ds(wid * (H_TROWS * HCHUNK) + t * HCHUNK, HCHUNK)])

        @pl.when(wid < 16)
        def _():
            pltpu.sync_copy(qe_r.at[wid], qidx_v)
            pltpu.sync_copy(ent_r.at[qidx_v.at[0]], qbuf_v)
            pltpu.sync_copy(qbuf_v, qe_o.at[pl.ds(wid * 8, 8)])

        @pl.when((wid >= 16) & (wid < 32))
        def _():
            w = wid - 16
            pltpu.sync_copy(qr_r.at[w], qidx_v)
            pltpu.sync_copy(rel_r.at[qidx_v.at[0]], qbuf_v)
            pltpu.sync_copy(qbuf_v, qr_o.at[pl.ds(w * 8, 8)])

    return k(ent, rel, nid3, qe3, qr3)


def _edge_kernel(ent_h, rel_h, h_h, src3, dstc4, typ3, qrl3, qnt3,
                 zacc, ones128):
    out_type = [
        jax.ShapeDtypeStruct((NC, ACC_ROWS, D), _f32),
        jax.ShapeDtypeStruct((NC, ACC_ROWS, D), _f32),
    ]
    scratch = [
        pltpu.VMEM((E_CH, EK), jnp.int32),
        pltpu.VMEM((E_CH, EK), jnp.int32),
        pltpu.VMEM((E_CH, EK), jnp.int32),
        pltpu.VMEM((E_CH, EK), jnp.int32),
        pltpu.VMEM((E_CH, EK), jnp.int32),
        pltpu.VMEM((EK, D), _f32),
        pltpu.VMEM((EK, D), _f32),
        pltpu.VMEM_SHARED((ACC_ROWS, D), _f32),
    ]

    @functools.partial(pl.kernel, out_type=out_type, mesh=_mesh(),
                       scratch_types=scratch)
    def k(ent_r, rel_r, h_r, src_r, dst_r, typ_r, qrl_r, qnt_r,
          zacc_r, ones_r,
          pre_o, deg_o, src_v, dst_v, typ_v, qrl_v, qnt_v, rows_v,
          ones_v, acc_s):
        c = lax.axis_index("c")
        s = lax.axis_index("s")

        def zero_acc():
            pltpu.sync_copy(zacc_r.at[pl.ds(s * ZROWS, ZROWS)],
                            acc_s.at[pl.ds(s * ZROWS, ZROWS)])

            @pl.when(s == 0)
            def _():
                pltpu.sync_copy(zacc_r.at[pl.ds(NS * ZROWS, ZTAIL)],
                                acc_s.at[pl.ds(NS * ZROWS, ZTAIL)])

        def write_acc(out):
            pltpu.sync_copy(acc_s.at[pl.ds(s * ZROWS, ZROWS)],
                            out.at[c, pl.ds(s * ZROWS, ZROWS)])

            @pl.when(s == 0)
            def _():
                pltpu.sync_copy(acc_s.at[pl.ds(NS * ZROWS, ZTAIL)],
                                out.at[c, pl.ds(NS * ZROWS, ZTAIL)])

        zero_acc()
        pltpu.sync_copy(ones_r, ones_v)

        plsc.subcore_barrier()

        @pl.loop(0, E_NCH)
        def _(cc):
            pltpu.sync_copy(src_r.at[s, pl.ds(cc * E_CH, E_CH)], src_v)
            pltpu.sync_copy(dst_r.at[c, s, pl.ds(cc * E_CH, E_CH)], dst_v)
            pltpu.sync_copy(typ_r.at[s, pl.ds(cc * E_CH, E_CH)], typ_v)
            pltpu.sync_copy(qrl_r.at[s, pl.ds(cc * E_CH, E_CH)], qrl_v)
            pltpu.sync_copy(qnt_r.at[s, pl.ds(cc * E_CH, E_CH)], qnt_v)

            @pl.loop(0, E_CH)
            def _(j):
                dst = dst_v.at[j]
                pltpu.sync_copy(ent_r.at[qnt_v.at[j]], rows_v)
                pltpu.sync_copy(rows_v, acc_s.at[dst], add=True)
                pltpu.sync_copy(h_r.at[src_v.at[j]], rows_v)
                pltpu.sync_copy(rows_v, acc_s.at[dst], add=True)
                pltpu.sync_copy(rel_r.at[typ_v.at[j]], rows_v)
                pltpu.sync_copy(rows_v, acc_s.at[dst], add=True)
                pltpu.sync_copy(rel_r.at[qrl_v.at[j]], rows_v)
                pltpu.sync_copy(rows_v, acc_s.at[dst], add=True)

        plsc.subcore_barrier()
        write_acc(pre_o)
        plsc.subcore_barrier()
        zero_acc()
        plsc.subcore_barrier()

        @pl.loop(0, E_NCH)
        def _(cc):
            pltpu.sync_copy(dst_r.at[c, s, pl.ds(cc * E_CH, E_CH)], dst_v)

            @pl.loop(0, E_CH)
            def _(j):
                pltpu.sync_copy(ones_v, acc_s.at[dst_v.at[j]], add=True)

        plsc.subcore_barrier()
        write_acc(deg_o)

    return k(ent_h, rel_h, h_h, src3, dstc4, typ3, qrl3, qnt3,
             zacc, ones128)


def _combine_tc(pre2, deg2, h, W_msg, W_self):
    R = NHALF

    def body(pre_r, deg_r, h_r, wm_r, ws_r, out_r):
        deg = deg_r[0, :, 0:1]
        agg = jnp.dot(pre_r[0], wm_r[...], preferred_element_type=_f32)
        agg = agg / jnp.maximum(deg, 1.0)
        hh = h_r[...]
        out_r[...] = jnp.maximum(
            jnp.dot(hh, ws_r[...], preferred_element_type=_f32) + agg, 0.0) + hh

    return pl.pallas_call(
        body,
        grid=(N_NODES // R,),
        in_specs=[
            pl.BlockSpec((1, R, D), lambda i: (i, 0, 0)),
            pl.BlockSpec((1, R, D), lambda i: (i, 0, 0)),
            pl.BlockSpec((R, D), lambda i: (i, 0)),
            pl.BlockSpec((D, D), lambda i: (0, 0)),
            pl.BlockSpec((D, D), lambda i: (0, 0)),
        ],
        out_specs=pl.BlockSpec((R, D), lambda i: (i, 0)),
        out_shape=jax.ShapeDtypeStruct((N_NODES, D), _f32),
    )(pre2, deg2, h, W_msg, W_self)


def _gather_history(total, bni3):
    @functools.partial(
        pl.kernel,
        out_type=jax.ShapeDtypeStruct((B_PAD, D), _f32),
        mesh=_mesh(),
        scratch_types=[
            pltpu.VMEM((B_TROWS, BCHUNK), jnp.int32),
            pltpu.VMEM((BCHUNK, D), _f32),
        ],
    )
    def k(tot_r, bni_r, out_o, idx_v, buf_v):
        c = lax.axis_index("c")
        s = lax.axis_index("s")
        wid = c * NS + s

        pltpu.sync_copy(bni_r.at[wid], idx_v)

        @pl.loop(0, B_TROWS)
        def _(t):
            pltpu.sync_copy(tot_r.at[idx_v.at[t]], buf_v)
            pltpu.sync_copy(
                buf_v,
                out_o.at[pl.ds(wid * (B_TROWS * BCHUNK) + t * BCHUNK, BCHUNK)])

    return k(total, bni3)


def kernel(ent_embeds, rel_embeds, W_msg, W_self, history_times,
           query_entities, query_relations, node_ids,
           edge_src, edge_dst, edge_type, edge_query_rel, edge_query_ent,
           batch_node_ids):
    ii = jnp.int32
    nid3 = jnp.pad(node_ids.astype(ii),
                   (0, H_PAD - N_NODES)).reshape(NW, H_TROWS, HCHUNK)
    qe3 = jnp.pad(query_entities.astype(ii), (0, 128 - BS)).reshape(16, 1, 8)
    qr3 = jnp.pad(query_relations.astype(ii), (0, 128 - BS)).reshape(16, 1, 8)

    epad = E_PAD - N_EDGES
    src3 = jnp.pad(edge_src.astype(ii), (0, epad)).reshape(NS, E_TROWS, EK)
    typ3 = jnp.pad(edge_type.astype(ii), (0, epad)).reshape(NS, E_TROWS, EK)
    qrl3 = jnp.pad(edge_query_rel.astype(ii),
                   (0, epad)).reshape(NS, E_TROWS, EK)
    qnt3 = jnp.pad(edge_query_ent.astype(ii),
                   (0, epad)).reshape(NS, E_TROWS, EK)
    dst_p = jnp.pad(edge_dst.astype(ii), (0, epad),
                    constant_values=N_NODES)
    dst0 = jnp.where(dst_p < NHALF, dst_p, NHALF)
    d1 = dst_p - NHALF
    dst1 = jnp.where((d1 >= 0) & (d1 < NHALF), d1, NHALF)
    dstc4 = jnp.stack([dst0, dst1]).reshape(NC, NS, E_TROWS, EK)

    bni3 = jnp.pad(batch_node_ids.astype(ii),
                   (0, B_PAD - BS * HIST)).reshape(NW, B_TROWS, BCHUNK)
    zacc = jnp.zeros((ACC_ROWS, D), _f32)
    ones128 = jnp.ones((EK, D), _f32)

    h, qe_pad, qr_pad = _gather_nodes_queries(ent_embeds, rel_embeds,
                                              nid3, qe3, qr3)

    pre2, deg2 = _edge_kernel(ent_embeds, rel_embeds, h, src3, dstc4, typ3,
                              qrl3, qnt3, zacc, ones128)

    total = _combine_tc(pre2, deg2, h, W_msg, W_self)
    hist_pad = _gather_history(total, bni3)

    history_gh = hist_pad[:BS * HIST].reshape(BS, HIST, D)
    history_pad_mask = (history_times == -1.0)[:, None, :]
    local_type = node_ids.reshape(BS, BS)
    return (qe_pad[:BS], qr_pad[:BS], history_gh, history_pad_mask,
            total, local_type)


# speedup vs baseline: 1.8010x; 1.2093x over previous
import functools

import jax
import jax.numpy as jnp
from jax import lax
from jax.experimental import pallas as pl
from jax.experimental.pallas import tpu as pltpu
from jax.experimental.pallas import tpu_sc as plsc

NC, NS, NW = 2, 16, 32
N_ENT, N_REL, D = 100000, 500, 128
N_NODES, N_EDGES = 10000, 320000
BS, HIST = 100, 50

NHALF = N_NODES // NC
ACC_ROWS = NHALF + 8

EK = 128
E_TROWS = 160
E_CH = 8
E_NCH = E_TROWS // E_CH
E_PAD = NS * E_TROWS * EK

ZROWS = 312
ZTAIL = ACC_ROWS - NS * ZROWS

HCHUNK = 40
H_PAD = 10240
H_TROWS = H_PAD // NW // HCHUNK

BCHUNK = 40
B_PAD = 5120
B_TROWS = B_PAD // NW // BCHUNK

_f32 = jnp.float32


def _mesh():
    return plsc.VectorSubcoreMesh(core_axis_name="c", subcore_axis_name="s")


def _gather_nodes_queries(ent, rel, nid3, qe3, qr3):
    @functools.partial(
        pl.kernel,
        out_type=[
            jax.ShapeDtypeStruct((H_PAD, D), _f32),
            jax.ShapeDtypeStruct((128, D), _f32),
            jax.ShapeDtypeStruct((128, D), _f32),
        ],
        mesh=_mesh(),
        scratch_types=[
            pltpu.VMEM((H_TROWS, HCHUNK), jnp.int32),
            pltpu.VMEM((HCHUNK, D), _f32),
            pltpu.VMEM((1, 8), jnp.int32),
            pltpu.VMEM((8, D), _f32),
        ],
    )
    def k(ent_r, rel_r, nid_r, qe_r, qr_r, h_o, qe_o, qr_o,
          nidx_v, buf_v, qidx_v, qbuf_v):
        c = lax.axis_index("c")
        s = lax.axis_index("s")
        wid = c * NS + s

        pltpu.sync_copy(nid_r.at[wid], nidx_v)

        @pl.loop(0, H_TROWS)
        def _(t):
            pltpu.sync_copy(ent_r.at[nidx_v.at[t]], buf_v)
            pltpu.sync_copy(
                buf_v, h_o.at[pl.ds(wid * (H_TROWS * HCHUNK) + t * HCHUNK, HCHUNK)])

        @pl.when(wid < 16)
        def _():
            pltpu.sync_copy(qe_r.at[wid], qidx_v)
            pltpu.sync_copy(ent_r.at[qidx_v.at[0]], qbuf_v)
            pltpu.sync_copy(qbuf_v, qe_o.at[pl.ds(wid * 8, 8)])

        @pl.when((wid >= 16) & (wid < 32))
        def _():
            w = wid - 16
            pltpu.sync_copy(qr_r.at[w], qidx_v)
            pltpu.sync_copy(rel_r.at[qidx_v.at[0]], qbuf_v)
            pltpu.sync_copy(qbuf_v, qr_o.at[pl.ds(w * 8, 8)])

    return k(ent, rel, nid3, qe3, qr3)


def _edge_kernel(ent_h, rel_h, h_h, src3, dstc4, typ3, qrl3, qnt3,
                 zacc, ones128):
    out_type = [
        jax.ShapeDtypeStruct((NC, ACC_ROWS, D), _f32),
        jax.ShapeDtypeStruct((NC, ACC_ROWS, D), _f32),
    ]
    scratch = [
        pltpu.VMEM((E_CH, EK), jnp.int32),
        pltpu.VMEM((E_CH, EK), jnp.int32),
        pltpu.VMEM((E_CH, EK), jnp.int32),
        pltpu.VMEM((E_CH, EK), jnp.int32),
        pltpu.VMEM((E_CH, EK), jnp.int32),
        pltpu.VMEM((EK, D), _f32),
        pltpu.VMEM((EK, D), _f32),
        pltpu.VMEM((EK, D), _f32),
        pltpu.VMEM((EK, D), _f32),
        pltpu.VMEM_SHARED((ACC_ROWS, D), _f32),
        pltpu.SemaphoreType.DMA,
        pltpu.SemaphoreType.DMA,
    ]

    @functools.partial(pl.kernel, out_type=out_type, mesh=_mesh(),
                       scratch_types=scratch)
    def k(ent_r, rel_r, h_r, src_r, dst_r, typ_r, qrl_r, qnt_r,
          zacc_r, ones_r,
          pre_o, deg_o, src_v, dst_v, typ_v, qrl_v, qnt_v,
          ra_v, rb_v, rc_v, rd_v, acc_s, gsem, ssem):
        c = lax.axis_index("c")
        s = lax.axis_index("s")

        def zero_acc():
            pltpu.sync_copy(zacc_r.at[pl.ds(s * ZROWS, ZROWS)],
                            acc_s.at[pl.ds(s * ZROWS, ZROWS)])

            @pl.when(s == 0)
            def _():
                pltpu.sync_copy(zacc_r.at[pl.ds(NS * ZROWS, ZTAIL)],
                                acc_s.at[pl.ds(NS * ZROWS, ZTAIL)])

        def write_acc(out):
            pltpu.sync_copy(acc_s.at[pl.ds(s * ZROWS, ZROWS)],
                            out.at[c, pl.ds(s * ZROWS, ZROWS)])

            @pl.when(s == 0)
            def _():
                pltpu.sync_copy(acc_s.at[pl.ds(NS * ZROWS, ZTAIL)],
                                out.at[c, pl.ds(NS * ZROWS, ZTAIL)])

        zero_acc()

        plsc.subcore_barrier()

        @pl.loop(0, E_NCH)
        def _(cc):
            pltpu.sync_copy(src_r.at[s, pl.ds(cc * E_CH, E_CH)], src_v)
            pltpu.sync_copy(dst_r.at[c, s, pl.ds(cc * E_CH, E_CH)], dst_v)
            pltpu.sync_copy(typ_r.at[s, pl.ds(cc * E_CH, E_CH)], typ_v)
            pltpu.sync_copy(qrl_r.at[s, pl.ds(cc * E_CH, E_CH)], qrl_v)
            pltpu.sync_copy(qnt_r.at[s, pl.ds(cc * E_CH, E_CH)], qnt_v)

            @pl.loop(0, E_CH)
            def _(j):
                dst = dst_v.at[j]
                gs = [pltpu.async_copy(ent_r.at[qnt_v.at[j]], ra_v, gsem),
                      pltpu.async_copy(h_r.at[src_v.at[j]], rb_v, gsem),
                      pltpu.async_copy(rel_r.at[typ_v.at[j]], rc_v, gsem),
                      pltpu.async_copy(rel_r.at[qrl_v.at[j]], rd_v, gsem)]
                for g in gs:
                    g.wait()
                ss = [pltpu.async_copy(ra_v, acc_s.at[dst], ssem, add=True),
                      pltpu.async_copy(rb_v, acc_s.at[dst], ssem, add=True),
                      pltpu.async_copy(rc_v, acc_s.at[dst], ssem, add=True),
                      pltpu.async_copy(rd_v, acc_s.at[dst], ssem, add=True)]
                for sh in ss:
                    sh.wait()

        plsc.subcore_barrier()
        write_acc(pre_o)
        plsc.subcore_barrier()
        zero_acc()
        plsc.subcore_barrier()

        pltpu.sync_copy(ones_r, ra_v)

        @pl.loop(0, E_NCH)
        def _(cc):
            pltpu.sync_copy(dst_r.at[c, s, pl.ds(cc * E_CH, E_CH)], dst_v)

            ss = [pltpu.async_copy(ra_v, acc_s.at[dst_v.at[j]], ssem,
                                   add=True)
                  for j in range(E_CH)]
            for sh in ss:
                sh.wait()

        plsc.subcore_barrier()
        write_acc(deg_o)

    return k(ent_h, rel_h, h_h, src3, dstc4, typ3, qrl3, qnt3,
             zacc, ones128)


def _combine_tc(pre2, deg2, h, W_msg, W_self):
    R = NHALF

    def body(pre_r, deg_r, h_r, wm_r, ws_r, out_r):
        deg = deg_r[0, :, 0:1]
        agg = jnp.dot(pre_r[0], wm_r[...], preferred_element_type=_f32)
        agg = agg / jnp.maximum(deg, 1.0)
        hh = h_r[...]
        out_r[...] = jnp.maximum(
            jnp.dot(hh, ws_r[...], preferred_element_type=_f32) + agg, 0.0) + hh

    return pl.pallas_call(
        body,
        grid=(N_NODES // R,),
        in_specs=[
            pl.BlockSpec((1, R, D), lambda i: (i, 0, 0)),
            pl.BlockSpec((1, R, D), lambda i: (i, 0, 0)),
            pl.BlockSpec((R, D), lambda i: (i, 0)),
            pl.BlockSpec((D, D), lambda i: (0, 0)),
            pl.BlockSpec((D, D), lambda i: (0, 0)),
        ],
        out_specs=pl.BlockSpec((R, D), lambda i: (i, 0)),
        out_shape=jax.ShapeDtypeStruct((N_NODES, D), _f32),
    )(pre2, deg2, h, W_msg, W_self)


def _gather_history(total, bni3):
    @functools.partial(
        pl.kernel,
        out_type=jax.ShapeDtypeStruct((B_PAD, D), _f32),
        mesh=_mesh(),
        scratch_types=[
            pltpu.VMEM((B_TROWS, BCHUNK), jnp.int32),
            pltpu.VMEM((BCHUNK, D), _f32),
        ],
    )
    def k(tot_r, bni_r, out_o, idx_v, buf_v):
        c = lax.axis_index("c")
        s = lax.axis_index("s")
        wid = c * NS + s

        pltpu.sync_copy(bni_r.at[wid], idx_v)

        @pl.loop(0, B_TROWS)
        def _(t):
            pltpu.sync_copy(tot_r.at[idx_v.at[t]], buf_v)
            pltpu.sync_copy(
                buf_v,
                out_o.at[pl.ds(wid * (B_TROWS * BCHUNK) + t * BCHUNK, BCHUNK)])

    return k(total, bni3)


def kernel(ent_embeds, rel_embeds, W_msg, W_self, history_times,
           query_entities, query_relations, node_ids,
           edge_src, edge_dst, edge_type, edge_query_rel, edge_query_ent,
           batch_node_ids):
    ii = jnp.int32
    nid3 = jnp.pad(node_ids.astype(ii),
                   (0, H_PAD - N_NODES)).reshape(NW, H_TROWS, HCHUNK)
    qe3 = jnp.pad(query_entities.astype(ii), (0, 128 - BS)).reshape(16, 1, 8)
    qr3 = jnp.pad(query_relations.astype(ii), (0, 128 - BS)).reshape(16, 1, 8)

    epad = E_PAD - N_EDGES
    src3 = jnp.pad(edge_src.astype(ii), (0, epad)).reshape(NS, E_TROWS, EK)
    typ3 = jnp.pad(edge_type.astype(ii), (0, epad)).reshape(NS, E_TROWS, EK)
    qrl3 = jnp.pad(edge_query_rel.astype(ii),
                   (0, epad)).reshape(NS, E_TROWS, EK)
    qnt3 = jnp.pad(edge_query_ent.astype(ii),
                   (0, epad)).reshape(NS, E_TROWS, EK)
    dst_p = jnp.pad(edge_dst.astype(ii), (0, epad),
                    constant_values=N_NODES)
    dst0 = jnp.where(dst_p < NHALF, dst_p, NHALF)
    d1 = dst_p - NHALF
    dst1 = jnp.where((d1 >= 0) & (d1 < NHALF), d1, NHALF)
    dstc4 = jnp.stack([dst0, dst1]).reshape(NC, NS, E_TROWS, EK)

    bni3 = jnp.pad(batch_node_ids.astype(ii),
                   (0, B_PAD - BS * HIST)).reshape(NW, B_TROWS, BCHUNK)
    zacc = jnp.zeros((ACC_ROWS, D), _f32)
    ones128 = jnp.ones((EK, D), _f32)

    h, qe_pad, qr_pad = _gather_nodes_queries(ent_embeds, rel_embeds,
                                              nid3, qe3, qr3)

    pre2, deg2 = _edge_kernel(ent_embeds, rel_embeds, h, src3, dstc4, typ3,
                              qrl3, qnt3, zacc, ones128)

    total = _combine_tc(pre2, deg2, h, W_msg, W_self)
    hist_pad = _gather_history(total, bni3)

    history_gh = hist_pad[:BS * HIST].reshape(BS, HIST, D)
    history_pad_mask = (history_times == -1.0)[:, None, :]
    local_type = node_ids.reshape(BS, BS)
    return (qe_pad[:BS], qr_pad[:BS], history_gh, history_pad_mask,
            total, local_type)
